# baseline (speedup 1.0000x reference)
import functools

import jax
import jax.numpy as jnp
from jax import lax
from jax.experimental import pallas as pl
from jax.experimental.pallas import tpu as pltpu
from jax.experimental.pallas import tpu_sc as plsc

_N = 100000
_E = 3200000
_EPS = 1e-5

_NP = 102400
_NP2 = (_NP // 128, 128)
_NW = 32
_EPW = 102400
_EP = _NW * _EPW
_W = 1024
_NWIN = _EPW // _W
_NCHUNK = _NP // 16

_mesh = plsc.VectorSubcoreMesh(core_axis_name="c", subcore_axis_name="s")
_sc_params = pltpu.CompilerParams(needs_layout_passes=False)


def _zero_acc(buf_v, acc_sh, sid):
    def zrow(i, _):
        buf_v[pl.ds(i * 16, 16)] = jnp.zeros((16,), jnp.float32)
        return 0

    lax.fori_loop(0, _NCHUNK // 16, zrow, 0)
    pltpu.sync_copy(buf_v, acc_sh.at[pl.ds(sid * _NCHUNK, _NCHUNK)])


def _write_out(buf_v, acc_sh, out_hbm, cid, sid):
    pltpu.sync_copy(acc_sh.at[pl.ds(sid * _NCHUNK, _NCHUNK)], buf_v)
    pltpu.sync_copy(buf_v, out_hbm.at[pl.ds(cid * _NP + sid * _NCHUNK, _NCHUNK)])


_NBUF = 4
_WD = 1024
_NWIN_D = _EPW // _WD


@functools.partial(
    pl.kernel,
    out_type=jax.ShapeDtypeStruct((2 * _NP,), jnp.float32),
    mesh=_mesh,
    compiler_params=_sc_params,
    scratch_types=(
        [pltpu.VMEM((_WD,), jnp.int32) for _ in range(_NBUF)]
        + [pltpu.VMEM((_WD,), jnp.float32) for _ in range(_NBUF)]
        + [pltpu.VMEM((_NCHUNK,), jnp.float32),
           pltpu.VMEM_SHARED((_NP,), jnp.float32)]
        + [pltpu.SemaphoreType.DMA for _ in range(2 * _NBUF)]
    ),
)
def _deg_pass(ei_hbm, ew_hbm, out_hbm,
              col0, col1, col2, col3, ew0, ew1, ew2, ew3, buf_v, acc_sh,
              si0, si1, si2, si3, ss0, ss1, ss2, ss3):
    cid = lax.axis_index("c")
    sid = lax.axis_index("s")
    wid = cid * 16 + sid

    cols = (col0, col1, col2, col3)
    ews = (ew0, ew1, ew2, ew3)
    sem_in = (si0, si1, si2, si3)
    sem_sc = (ss0, ss1, ss2, ss3)

    def valid(w):
        return wid * _EPW + w * _WD < _E

    def start_in(w):
        p = w % _NBUF
        base = wid * _EPW + w * _WD
        c1 = pltpu.async_copy(ei_hbm.at[1, pl.ds(base, _WD)], cols[p],
                              sem_in[p])
        c2 = pltpu.async_copy(ew_hbm.at[pl.ds(base, _WD)], ews[p], sem_in[p])
        return (c1, c2)

    def start_in_when(w):
        @pl.when(valid(w))
        def _():
            start_in(w)
        p = w % _NBUF
        return (pltpu.make_async_copy(ei_hbm.at[1, pl.ds(0, _WD)], cols[p],
                                      sem_in[p]),
                pltpu.make_async_copy(ew_hbm.at[pl.ds(0, _WD)], ews[p],
                                      sem_in[p]))

    ins = {0: start_in_when(0), 1: start_in_when(1)}
    scs = {}
    conds = {0: valid(0), 1: valid(1)}
    _zero_acc(buf_v, acc_sh, sid)
    plsc.subcore_barrier()
    for w in range(_NWIN_D):
        p = w % _NBUF
        if w - 2 in scs:
            sc_desc = scs.pop(w - 2)

            @pl.when(conds[w - 2])
            def _(sc_desc=sc_desc):
                sc_desc.wait()
        if w + 2 < _NWIN_D:
            conds[w + 2] = valid(w + 2)
            ins[w + 2] = start_in_when(w + 2)
        in_descs = ins.pop(w)
        cw = conds[w]

        @pl.when(cw)
        def _(in_descs=in_descs, p=p):
            for c in in_descs:
                c.wait()

        @pl.when(cw)
        def _(p=p):
            pltpu.async_copy(ews[p], acc_sh.at[cols[p]], sem_sc[p], add=True)
        scs[w] = pltpu.make_async_copy(ews[p], acc_sh.at[cols[p]], sem_sc[p])
    for w in list(scs):
        sc_desc = scs.pop(w)

        @pl.when(conds[w])
        def _(sc_desc=sc_desc):
            sc_desc.wait()
    plsc.subcore_barrier()
    _write_out(buf_v, acc_sh, out_hbm, cid, sid)


@functools.partial(
    pl.kernel,
    out_type=jax.ShapeDtypeStruct((2 * _NP,), jnp.float32),
    mesh=_mesh,
    compiler_params=_sc_params,
    scratch_types=(
        [pltpu.VMEM((_NP,), jnp.float32)]
        + [pltpu.VMEM((_W,), jnp.int32) for _ in range(_NBUF)]
        + [pltpu.VMEM((_W,), jnp.int32) for _ in range(_NBUF)]
        + [pltpu.VMEM((_W,), jnp.float32) for _ in range(_NBUF)]
        + [pltpu.VMEM_SHARED((_NP,), jnp.float32)]
        + [pltpu.SemaphoreType.DMA for _ in range(2 * _NBUF + 1)]
    ),
)
def _agg_pass(table_hbm, ei_hbm, ew_hbm, out_hbm,
              table_v, row0, row1, row2, row3, col0, col1, col2, col3,
              ew0, ew1, ew2, ew3, acc_sh,
              si0, si1, si2, si3, ss0, ss1, ss2, ss3, sem_t):
    cid = lax.axis_index("c")
    sid = lax.axis_index("s")
    wid = cid * 16 + sid

    rows = (row0, row1, row2, row3)
    cols = (col0, col1, col2, col3)
    ews = (ew0, ew1, ew2, ew3)
    sem_in = (si0, si1, si2, si3)
    sem_sc = (ss0, ss1, ss2, ss3)

    def valid(w):
        return wid * _EPW + w * _W < _E

    def start_in(w):
        p = w % _NBUF
        base = wid * _EPW + w * _W
        pltpu.async_copy(ei_hbm.at[0, pl.ds(base, _W)], rows[p], sem_in[p])
        pltpu.async_copy(ei_hbm.at[1, pl.ds(base, _W)], cols[p], sem_in[p])
        pltpu.async_copy(ew_hbm.at[pl.ds(base, _W)], ews[p], sem_in[p])

    def start_in_when(w):
        @pl.when(valid(w))
        def _():
            start_in(w)
        p = w % _NBUF
        return (pltpu.make_async_copy(ei_hbm.at[0, pl.ds(0, _W)], rows[p],
                                      sem_in[p]),
                pltpu.make_async_copy(ei_hbm.at[1, pl.ds(0, _W)], cols[p],
                                      sem_in[p]),
                pltpu.make_async_copy(ew_hbm.at[pl.ds(0, _W)], ews[p],
                                      sem_in[p]))

    tcopy = pltpu.async_copy(table_hbm, table_v, sem_t)
    conds = {0: valid(0), 1: valid(1)}
    ins = {0: start_in_when(0), 1: start_in_when(1)}
    zb = ews[2]

    def zrow(i, _):
        zb[pl.ds(i * 16, 16)] = jnp.zeros((16,), jnp.float32)
        return 0

    lax.fori_loop(0, _W // 16, zrow, 0)
    for q in range(_NCHUNK // _W):
        pltpu.sync_copy(zb, acc_sh.at[pl.ds(sid * _NCHUNK + q * _W, _W)])
    if _NCHUNK % _W:
        pltpu.sync_copy(zb.at[pl.ds(0, _NCHUNK % _W)],
                        acc_sh.at[pl.ds(sid * _NCHUNK + (_NCHUNK // _W) * _W,
                                        _NCHUNK % _W)])
    tcopy.wait()
    plsc.subcore_barrier()

    scs = {}
    for w in range(_NWIN):
        p = w % _NBUF
        if w - 2 in scs:
            sc_desc = scs.pop(w - 2)

            @pl.when(conds[w - 2])
            def _(sc_desc=sc_desc):
                sc_desc.wait()
        if w + 2 < _NWIN:
            conds[w + 2] = valid(w + 2)
            ins[w + 2] = start_in_when(w + 2)
        in_descs = ins.pop(w)
        cw = conds[w]

        @pl.when(cw)
        def _(in_descs=in_descs, p=p):
            for c in in_descs:
                c.wait()

            @plsc.parallel_loop(0, _W, 16, unroll=8)
            def seg(i):
                sl = pl.ds(i, 16)
                vals = plsc.load_gather(table_v, [rows[p][sl]])
                ews[p][sl] = ews[p][sl] * vals
            pltpu.async_copy(ews[p], acc_sh.at[cols[p]], sem_sc[p], add=True)
        scs[w] = pltpu.make_async_copy(ews[p], acc_sh.at[cols[p]], sem_sc[p])
    for w in list(scs):
        sc_desc = scs.pop(w)

        @pl.when(conds[w])
        def _(sc_desc=sc_desc):
            sc_desc.wait()
    plsc.subcore_barrier()
    for q in range(_NCHUNK // _W):
        pltpu.sync_copy(acc_sh.at[pl.ds(sid * _NCHUNK + q * _W, _W)], ew0)
        pltpu.sync_copy(
            ew0, out_hbm.at[pl.ds(cid * _NP + sid * _NCHUNK + q * _W, _W)])
    if _NCHUNK % _W:
        _r = _NCHUNK % _W
        _o = (_NCHUNK // _W) * _W
        pltpu.sync_copy(acc_sh.at[pl.ds(sid * _NCHUNK + _o, _r)],
                        ew0.at[pl.ds(0, _r)])
        pltpu.sync_copy(ew0.at[pl.ds(0, _r)],
                        out_hbm.at[pl.ds(cid * _NP + sid * _NCHUNK + _o, _r)])


@functools.partial(
    pl.kernel,
    out_type=jax.ShapeDtypeStruct((4 * 2 * _NP,), jnp.float32),
    mesh=_mesh,
    compiler_params=_sc_params,
    scratch_types=(
        [pltpu.VMEM((_NP,), jnp.float32)]
        + [pltpu.VMEM((_W,), jnp.int32) for _ in range(_NBUF)]
        + [pltpu.VMEM((_W,), jnp.int32) for _ in range(_NBUF)]
        + [pltpu.VMEM((_W,), jnp.float32) for _ in range(_NBUF)]
        + [pltpu.VMEM_SHARED((_NP,), jnp.float32)]
        + [pltpu.SemaphoreType.DMA for _ in range(2 * _NBUF + 1)]
    ),
)
def _l2_pass(tab4_hbm, ei_hbm, ew_hbm, out_hbm,
             table_v, row0, row1, row2, row3, col0, col1, col2, col3,
             ew0, ew1, ew2, ew3, acc_sh,
             si0, si1, si2, si3, ss0, ss1, ss2, ss3, sem_t):
    cid = lax.axis_index("c")
    sid = lax.axis_index("s")
    wid = cid * 16 + sid

    rows = (row0, row1, row2, row3)
    cols = (col0, col1, col2, col3)
    ews = (ew0, ew1, ew2, ew3)
    sem_in = (si0, si1, si2, si3)
    sem_sc = (ss0, ss1, ss2, ss3)

    def valid(w):
        return wid * _EPW + w * _W < _E

    def start_in(w):
        p = w % _NBUF
        base = wid * _EPW + w * _W
        pltpu.async_copy(ei_hbm.at[0, pl.ds(base, _W)], rows[p], sem_in[p])
        pltpu.async_copy(ei_hbm.at[1, pl.ds(base, _W)], cols[p], sem_in[p])
        pltpu.async_copy(ew_hbm.at[pl.ds(base, _W)], ews[p], sem_in[p])

    def start_in_when(w):
        @pl.when(valid(w))
        def _():
            start_in(w)
        p = w % _NBUF
        return (pltpu.make_async_copy(ei_hbm.at[0, pl.ds(0, _W)], rows[p],
                                      sem_in[p]),
                pltpu.make_async_copy(ei_hbm.at[1, pl.ds(0, _W)], cols[p],
                                      sem_in[p]),
                pltpu.make_async_copy(ew_hbm.at[pl.ds(0, _W)], ews[p],
                                      sem_in[p]))

    def zero_acc(zb):
        def zrow(i, _):
            zb[pl.ds(i * 16, 16)] = jnp.zeros((16,), jnp.float32)
            return 0

        lax.fori_loop(0, _W // 16, zrow, 0)
        for q in range(_NCHUNK // _W):
            pltpu.sync_copy(zb, acc_sh.at[pl.ds(sid * _NCHUNK + q * _W, _W)])
        if _NCHUNK % _W:
            pltpu.sync_copy(
                zb.at[pl.ds(0, _NCHUNK % _W)],
                acc_sh.at[pl.ds(sid * _NCHUNK + (_NCHUNK // _W) * _W,
                                _NCHUNK % _W)])

    def wait_table():
        pltpu.make_async_copy(tab4_hbm.at[0], table_v, sem_t).wait()

    pltpu.async_copy(tab4_hbm.at[0], table_v, sem_t)
    zero_acc(ews[2])
    plsc.subcore_barrier()

    def chan_body(ch, _):
        wait_table()
        conds = {0: valid(0), 1: valid(1)}
        ins = {0: start_in_when(0), 1: start_in_when(1)}
        scs = {}
        for w in range(_NWIN):
            p = w % _NBUF
            if w - 2 in scs:
                sc_desc = scs.pop(w - 2)

                @pl.when(conds[w - 2])
                def _(sc_desc=sc_desc):
                    sc_desc.wait()
            if w + 2 < _NWIN:
                conds[w + 2] = valid(w + 2)
                ins[w + 2] = start_in_when(w + 2)
            in_descs = ins.pop(w)
            cw = conds[w]

            @pl.when(cw)
            def _(in_descs=in_descs, p=p):
                for c in in_descs:
                    c.wait()

                @plsc.parallel_loop(0, _W, 16, unroll=8)
                def seg(i):
                    sl = pl.ds(i, 16)
                    vals = plsc.load_gather(table_v, [rows[p][sl]])
                    ews[p][sl] = ews[p][sl] * vals
                pltpu.async_copy(ews[p], acc_sh.at[cols[p]], sem_sc[p],
                                 add=True)
            scs[w] = pltpu.make_async_copy(ews[p], acc_sh.at[cols[p]],
                                           sem_sc[p])
        for w in list(scs):
            sc_desc = scs.pop(w)

            @pl.when(conds[w])
            def _(sc_desc=sc_desc):
                sc_desc.wait()
        plsc.subcore_barrier()
        pltpu.async_copy(tab4_hbm.at[jnp.minimum(ch + 1, 3)], table_v, sem_t)
        zb = ews[2]
        obase = ch * (2 * _NP) + cid * _NP + sid * _NCHUNK
        for q in range(_NCHUNK // _W):
            pltpu.sync_copy(acc_sh.at[pl.ds(sid * _NCHUNK + q * _W, _W)], zb)
            pltpu.sync_copy(zb, out_hbm.at[pl.ds(obase + q * _W, _W)])
        if _NCHUNK % _W:
            _r = _NCHUNK % _W
            _o = (_NCHUNK // _W) * _W
            pltpu.sync_copy(acc_sh.at[pl.ds(sid * _NCHUNK + _o, _r)],
                            zb.at[pl.ds(0, _r)])
            pltpu.sync_copy(zb.at[pl.ds(0, _r)],
                            out_hbm.at[pl.ds(obase + _o, _r)])
        zero_acc(zb)
        plsc.subcore_barrier()
        return 0

    lax.fori_loop(0, 4, chan_body, 0)
    wait_table()


def _d1_body(degp_ref, x_ref, dinv_ref, a1_ref):
    deg = degp_ref[0] + degp_ref[1] + 1.0
    dinv = lax.rsqrt(deg)
    dinv_ref[...] = dinv
    a1_ref[...] = dinv * x_ref[...]


_d1 = pl.pallas_call(
    _d1_body,
    out_shape=[jax.ShapeDtypeStruct(_NP2, jnp.float32),
               jax.ShapeDtypeStruct(_NP2, jnp.float32)],
)


def _d2_body(gp_ref, a1_ref, dinv_ref, c1_ref, d1_ref, w2_ref, a2t_ref):
    dinv = dinv_ref[...]
    s1 = dinv * (gp_ref[0] + gp_ref[1] + a1_ref[...])
    acc = [jnp.zeros(_NP2, jnp.float32) for _ in range(4)]
    for k in range(16):
        h = jnp.maximum(s1 * c1_ref[k] + d1_ref[k], 0.0)
        for j in range(4):
            acc[j] = acc[j] + h * w2_ref[k, j]
    for j in range(4):
        a2t_ref[j] = dinv * acc[j]


_d2 = pl.pallas_call(
    _d2_body,
    in_specs=[
        pl.BlockSpec(memory_space=pltpu.VMEM),
        pl.BlockSpec(memory_space=pltpu.VMEM),
        pl.BlockSpec(memory_space=pltpu.VMEM),
        pl.BlockSpec(memory_space=pltpu.SMEM),
        pl.BlockSpec(memory_space=pltpu.SMEM),
        pl.BlockSpec(memory_space=pltpu.SMEM),
    ],
    out_shape=jax.ShapeDtypeStruct((4,) + _NP2, jnp.float32),
)


def _d3_body(gp_ref, a2t_ref, dinv_ref, c2_ref, d2_ref, w3_ref, a3_ref):
    dinv = dinv_ref[...]
    u = jnp.zeros(_NP2, jnp.float32)
    for j in range(4):
        s2 = dinv * (gp_ref[j, 0] + gp_ref[j, 1] + a2t_ref[j])
        h2 = jnp.maximum(s2 * c2_ref[j] + d2_ref[j], 0.0)
        u = u + h2 * w3_ref[j]
    a3_ref[...] = dinv * u


_d3 = pl.pallas_call(
    _d3_body,
    in_specs=[
        pl.BlockSpec(memory_space=pltpu.VMEM),
        pl.BlockSpec(memory_space=pltpu.VMEM),
        pl.BlockSpec(memory_space=pltpu.VMEM),
        pl.BlockSpec(memory_space=pltpu.SMEM),
        pl.BlockSpec(memory_space=pltpu.SMEM),
        pl.BlockSpec(memory_space=pltpu.SMEM),
    ],
    out_shape=jax.ShapeDtypeStruct(_NP2, jnp.float32),
)


def _d4_body(gp_ref, a3_ref, dinv_ref, ab_ref, z_ref):
    s3 = dinv_ref[...] * (gp_ref[0] + gp_ref[1] + a3_ref[...])
    z_ref[...] = jax.nn.sigmoid(s3 * ab_ref[0] + ab_ref[1])


_d4 = pl.pallas_call(
    _d4_body,
    in_specs=[
        pl.BlockSpec(memory_space=pltpu.VMEM),
        pl.BlockSpec(memory_space=pltpu.VMEM),
        pl.BlockSpec(memory_space=pltpu.VMEM),
        pl.BlockSpec(memory_space=pltpu.SMEM),
    ],
    out_shape=jax.ShapeDtypeStruct(_NP2, jnp.float32),
)


def kernel(x, edge_attr, W1, b1, g1, be1, W2, b2, g2, be2, W3, b3, g3, be3,
           lw, lb, edge_index):
    f32 = jnp.float32
    ewp = edge_attr
    xp = jnp.concatenate([x[:, 0], jnp.zeros((_NP - _N,), f32)]).reshape(_NP2)

    k1 = f32(1.0) / jnp.sqrt(f32(1.0 + _EPS))
    c1 = W1[0] * k1 * g1
    d1 = b1 * k1 * g1 + be1
    c2 = k1 * g2
    d2 = b2 * k1 * g2 + be2
    w3 = W3[:, 0]
    A = k1 * g3[0] * lw[0, 0]
    B = (b3[0] * k1 * g3[0] + be3[0]) * lw[0, 0] + lb[0]
    ab = jnp.stack([A, B])

    degp = _deg_pass(edge_index, ewp).reshape((2,) + _NP2)
    dinv2, a1_2 = _d1(degp, xp)
    g1p = _agg_pass(a1_2.reshape(_NP), edge_index, ewp).reshape((2,) + _NP2)
    a2t = _d2(g1p, a1_2, dinv2, c1, d1, W2)
    g2p = _l2_pass(a2t.reshape(4, _NP), edge_index, ewp).reshape(
        (4, 2) + _NP2)
    a3_2 = _d3(g2p, a2t, dinv2, c2, d2, w3)
    g3p = _agg_pass(a3_2.reshape(_NP), edge_index, ewp).reshape((2,) + _NP2)
    z = _d4(g3p, a3_2, dinv2, ab)
    return z.reshape(_NP)[:_N].reshape(_N, 1)


# speedup vs baseline: 1.0861x; 1.0861x over previous
import functools

import jax
import jax.numpy as jnp
from jax import lax
from jax.experimental import pallas as pl
from jax.experimental.pallas import tpu as pltpu
from jax.experimental.pallas import tpu_sc as plsc

_N = 100000
_E = 3200000
_EPS = 1e-5

_NP = 102400
_NP2 = (_NP // 128, 128)
_NW = 32
_EPW = 102400
_EP = _NW * _EPW
_W = 1024
_NWIN = _EPW // _W
_NCHUNK = _NP // 16

_mesh = plsc.VectorSubcoreMesh(core_axis_name="c", subcore_axis_name="s")
_sc_params = pltpu.CompilerParams(needs_layout_passes=False)


def _zero_acc(buf_v, acc_sh, sid):
    def zrow(i, _):
        buf_v[pl.ds(i * 16, 16)] = jnp.zeros((16,), jnp.float32)
        return 0

    lax.fori_loop(0, _NCHUNK // 16, zrow, 0)
    pltpu.sync_copy(buf_v, acc_sh.at[pl.ds(sid * _NCHUNK, _NCHUNK)])


def _write_out(buf_v, acc_sh, out_hbm, cid, sid):
    pltpu.sync_copy(acc_sh.at[pl.ds(sid * _NCHUNK, _NCHUNK)], buf_v)
    pltpu.sync_copy(buf_v, out_hbm.at[pl.ds(cid * _NP + sid * _NCHUNK, _NCHUNK)])


_NBUF = 6
_LAG = _NBUF - 2
_WD = 1024
_NWIN_D = _EPW // _WD


@functools.partial(
    pl.kernel,
    out_type=jax.ShapeDtypeStruct((2 * _NP,), jnp.float32),
    mesh=_mesh,
    compiler_params=_sc_params,
    scratch_types=(
        [pltpu.VMEM((_WD,), jnp.int32) for _ in range(_NBUF)]
        + [pltpu.VMEM((_WD,), jnp.float32) for _ in range(_NBUF)]
        + [pltpu.VMEM((_NCHUNK,), jnp.float32),
           pltpu.VMEM_SHARED((_NP,), jnp.float32)]
        + [pltpu.SemaphoreType.DMA for _ in range(2 * _NBUF)]
    ),
)
def _deg_pass(ei_hbm, ew_hbm, out_hbm,
              col0, col1, col2, col3, col4, col5,
              ew0, ew1, ew2, ew3, ew4, ew5, buf_v, acc_sh,
              si0, si1, si2, si3, si4, si5, ss0, ss1, ss2, ss3, ss4, ss5):
    cid = lax.axis_index("c")
    sid = lax.axis_index("s")
    wid = cid * 16 + sid

    cols = (col0, col1, col2, col3, col4, col5)
    ews = (ew0, ew1, ew2, ew3, ew4, ew5)
    sem_in = (si0, si1, si2, si3, si4, si5)
    sem_sc = (ss0, ss1, ss2, ss3, ss4, ss5)

    def valid(w):
        return wid * _EPW + w * _WD < _E

    def start_in(w):
        p = w % _NBUF
        base = wid * _EPW + w * _WD
        c1 = pltpu.async_copy(ei_hbm.at[1, pl.ds(base, _WD)], cols[p],
                              sem_in[p])
        c2 = pltpu.async_copy(ew_hbm.at[pl.ds(base, _WD)], ews[p], sem_in[p])
        return (c1, c2)

    def start_in_when(w):
        @pl.when(valid(w))
        def _():
            start_in(w)
        p = w % _NBUF
        return (pltpu.make_async_copy(ei_hbm.at[1, pl.ds(0, _WD)], cols[p],
                                      sem_in[p]),
                pltpu.make_async_copy(ew_hbm.at[pl.ds(0, _WD)], ews[p],
                                      sem_in[p]))

    ins = {0: start_in_when(0), 1: start_in_when(1)}
    scs = {}
    conds = {0: valid(0), 1: valid(1)}
    _zero_acc(buf_v, acc_sh, sid)
    plsc.subcore_barrier()
    for w in range(_NWIN_D):
        p = w % _NBUF
        if w - _LAG in scs:
            sc_desc = scs.pop(w - _LAG)

            @pl.when(conds[w - _LAG])
            def _(sc_desc=sc_desc):
                sc_desc.wait()
        if w + 2 < _NWIN_D:
            conds[w + 2] = valid(w + 2)
            ins[w + 2] = start_in_when(w + 2)
        in_descs = ins.pop(w)
        cw = conds[w]

        @pl.when(cw)
        def _(in_descs=in_descs, p=p):
            for c in in_descs:
                c.wait()

        @pl.when(cw)
        def _(p=p):
            pltpu.async_copy(ews[p], acc_sh.at[cols[p]], sem_sc[p], add=True)
        scs[w] = pltpu.make_async_copy(ews[p], acc_sh.at[cols[p]], sem_sc[p])
    for w in list(scs):
        sc_desc = scs.pop(w)

        @pl.when(conds[w])
        def _(sc_desc=sc_desc):
            sc_desc.wait()
    plsc.subcore_barrier()
    _write_out(buf_v, acc_sh, out_hbm, cid, sid)


@functools.partial(
    pl.kernel,
    out_type=jax.ShapeDtypeStruct((2 * _NP,), jnp.float32),
    mesh=_mesh,
    compiler_params=_sc_params,
    scratch_types=(
        [pltpu.VMEM((_NP,), jnp.float32)]
        + [pltpu.VMEM((_W,), jnp.int32) for _ in range(_NBUF)]
        + [pltpu.VMEM((_W,), jnp.int32) for _ in range(_NBUF)]
        + [pltpu.VMEM((_W,), jnp.float32) for _ in range(_NBUF)]
        + [pltpu.VMEM_SHARED((_NP,), jnp.float32)]
        + [pltpu.SemaphoreType.DMA for _ in range(2 * _NBUF + 1)]
    ),
)
def _agg_pass(table_hbm, ei_hbm, ew_hbm, out_hbm,
              table_v, row0, row1, row2, row3, row4, row5,
              col0, col1, col2, col3, col4, col5,
              ew0, ew1, ew2, ew3, ew4, ew5, acc_sh,
              si0, si1, si2, si3, si4, si5,
              ss0, ss1, ss2, ss3, ss4, ss5, sem_t):
    cid = lax.axis_index("c")
    sid = lax.axis_index("s")
    wid = cid * 16 + sid

    rows = (row0, row1, row2, row3, row4, row5)
    cols = (col0, col1, col2, col3, col4, col5)
    ews = (ew0, ew1, ew2, ew3, ew4, ew5)
    sem_in = (si0, si1, si2, si3, si4, si5)
    sem_sc = (ss0, ss1, ss2, ss3, ss4, ss5)

    def valid(w):
        return wid * _EPW + w * _W < _E

    def start_in(w):
        p = w % _NBUF
        base = wid * _EPW + w * _W
        pltpu.async_copy(ei_hbm.at[0, pl.ds(base, _W)], rows[p], sem_in[p])
        pltpu.async_copy(ei_hbm.at[1, pl.ds(base, _W)], cols[p], sem_in[p])
        pltpu.async_copy(ew_hbm.at[pl.ds(base, _W)], ews[p], sem_in[p])

    def start_in_when(w):
        @pl.when(valid(w))
        def _():
            start_in(w)
        p = w % _NBUF
        return (pltpu.make_async_copy(ei_hbm.at[0, pl.ds(0, _W)], rows[p],
                                      sem_in[p]),
                pltpu.make_async_copy(ei_hbm.at[1, pl.ds(0, _W)], cols[p],
                                      sem_in[p]),
                pltpu.make_async_copy(ew_hbm.at[pl.ds(0, _W)], ews[p],
                                      sem_in[p]))

    tcopy = pltpu.async_copy(table_hbm, table_v, sem_t)
    conds = {0: valid(0), 1: valid(1)}
    ins = {0: start_in_when(0), 1: start_in_when(1)}
    zb = ews[2]

    def zrow(i, _):
        zb[pl.ds(i * 16, 16)] = jnp.zeros((16,), jnp.float32)
        return 0

    lax.fori_loop(0, _W // 16, zrow, 0)
    for q in range(_NCHUNK // _W):
        pltpu.sync_copy(zb, acc_sh.at[pl.ds(sid * _NCHUNK + q * _W, _W)])
    if _NCHUNK % _W:
        pltpu.sync_copy(zb.at[pl.ds(0, _NCHUNK % _W)],
                        acc_sh.at[pl.ds(sid * _NCHUNK + (_NCHUNK // _W) * _W,
                                        _NCHUNK % _W)])
    tcopy.wait()
    plsc.subcore_barrier()

    scs = {}
    for w in range(_NWIN):
        p = w % _NBUF
        if w - _LAG in scs:
            sc_desc = scs.pop(w - _LAG)

            @pl.when(conds[w - _LAG])
            def _(sc_desc=sc_desc):
                sc_desc.wait()
        if w + 2 < _NWIN:
            conds[w + 2] = valid(w + 2)
            ins[w + 2] = start_in_when(w + 2)
        in_descs = ins.pop(w)
        cw = conds[w]

        @pl.when(cw)
        def _(in_descs=in_descs, p=p):
            for c in in_descs:
                c.wait()

            @plsc.parallel_loop(0, _W, 16, unroll=4)
            def seg(i):
                sl = pl.ds(i, 16)
                vals = plsc.load_gather(table_v, [rows[p][sl]])
                ews[p][sl] = ews[p][sl] * vals
            pltpu.async_copy(ews[p], acc_sh.at[cols[p]], sem_sc[p], add=True)
        scs[w] = pltpu.make_async_copy(ews[p], acc_sh.at[cols[p]], sem_sc[p])
    for w in list(scs):
        sc_desc = scs.pop(w)

        @pl.when(conds[w])
        def _(sc_desc=sc_desc):
            sc_desc.wait()
    plsc.subcore_barrier()
    for q in range(_NCHUNK // _W):
        pltpu.sync_copy(acc_sh.at[pl.ds(sid * _NCHUNK + q * _W, _W)], ew0)
        pltpu.sync_copy(
            ew0, out_hbm.at[pl.ds(cid * _NP + sid * _NCHUNK + q * _W, _W)])
    if _NCHUNK % _W:
        _r = _NCHUNK % _W
        _o = (_NCHUNK // _W) * _W
        pltpu.sync_copy(acc_sh.at[pl.ds(sid * _NCHUNK + _o, _r)],
                        ew0.at[pl.ds(0, _r)])
        pltpu.sync_copy(ew0.at[pl.ds(0, _r)],
                        out_hbm.at[pl.ds(cid * _NP + sid * _NCHUNK + _o, _r)])


@functools.partial(
    pl.kernel,
    out_type=jax.ShapeDtypeStruct((4 * 2 * _NP,), jnp.float32),
    mesh=_mesh,
    compiler_params=_sc_params,
    scratch_types=(
        [pltpu.VMEM((_NP,), jnp.float32)]
        + [pltpu.VMEM((_W,), jnp.int32) for _ in range(_NBUF)]
        + [pltpu.VMEM((_W,), jnp.int32) for _ in range(_NBUF)]
        + [pltpu.VMEM((_W,), jnp.float32) for _ in range(_NBUF)]
        + [pltpu.VMEM_SHARED((_NP,), jnp.float32)]
        + [pltpu.SemaphoreType.DMA for _ in range(2 * _NBUF + 1)]
    ),
)
def _l2_pass(tab4_hbm, ei_hbm, ew_hbm, out_hbm,
             table_v, row0, row1, row2, row3, row4, row5,
             col0, col1, col2, col3, col4, col5,
             ew0, ew1, ew2, ew3, ew4, ew5, acc_sh,
             si0, si1, si2, si3, si4, si5,
             ss0, ss1, ss2, ss3, ss4, ss5, sem_t):
    cid = lax.axis_index("c")
    sid = lax.axis_index("s")
    wid = cid * 16 + sid

    rows = (row0, row1, row2, row3, row4, row5)
    cols = (col0, col1, col2, col3, col4, col5)
    ews = (ew0, ew1, ew2, ew3, ew4, ew5)
    sem_in = (si0, si1, si2, si3, si4, si5)
    sem_sc = (ss0, ss1, ss2, ss3, ss4, ss5)

    def valid(w):
        return wid * _EPW + w * _W < _E

    def start_in(w):
        p = w % _NBUF
        base = wid * _EPW + w * _W
        pltpu.async_copy(ei_hbm.at[0, pl.ds(base, _W)], rows[p], sem_in[p])
        pltpu.async_copy(ei_hbm.at[1, pl.ds(base, _W)], cols[p], sem_in[p])
        pltpu.async_copy(ew_hbm.at[pl.ds(base, _W)], ews[p], sem_in[p])

    def start_in_when(w):
        @pl.when(valid(w))
        def _():
            start_in(w)
        p = w % _NBUF
        return (pltpu.make_async_copy(ei_hbm.at[0, pl.ds(0, _W)], rows[p],
                                      sem_in[p]),
                pltpu.make_async_copy(ei_hbm.at[1, pl.ds(0, _W)], cols[p],
                                      sem_in[p]),
                pltpu.make_async_copy(ew_hbm.at[pl.ds(0, _W)], ews[p],
                                      sem_in[p]))

    def zero_acc(zb):
        def zrow(i, _):
            zb[pl.ds(i * 16, 16)] = jnp.zeros((16,), jnp.float32)
            return 0

        lax.fori_loop(0, _W // 16, zrow, 0)
        for q in range(_NCHUNK // _W):
            pltpu.sync_copy(zb, acc_sh.at[pl.ds(sid * _NCHUNK + q * _W, _W)])
        if _NCHUNK % _W:
            pltpu.sync_copy(
                zb.at[pl.ds(0, _NCHUNK % _W)],
                acc_sh.at[pl.ds(sid * _NCHUNK + (_NCHUNK // _W) * _W,
                                _NCHUNK % _W)])

    def wait_table():
        pltpu.make_async_copy(tab4_hbm.at[0], table_v, sem_t).wait()

    pltpu.async_copy(tab4_hbm.at[0], table_v, sem_t)
    zero_acc(ews[2])
    plsc.subcore_barrier()

    def chan_body(ch, _):
        wait_table()
        conds = {0: valid(0), 1: valid(1)}
        ins = {0: start_in_when(0), 1: start_in_when(1)}
        scs = {}
        for w in range(_NWIN):
            p = w % _NBUF
            if w - _LAG in scs:
                sc_desc = scs.pop(w - _LAG)

                @pl.when(conds[w - _LAG])
                def _(sc_desc=sc_desc):
                    sc_desc.wait()
            if w + 2 < _NWIN:
                conds[w + 2] = valid(w + 2)
                ins[w + 2] = start_in_when(w + 2)
            in_descs = ins.pop(w)
            cw = conds[w]

            @pl.when(cw)
            def _(in_descs=in_descs, p=p):
                for c in in_descs:
                    c.wait()

                @plsc.parallel_loop(0, _W, 16, unroll=4)
                def seg(i):
                    sl = pl.ds(i, 16)
                    vals = plsc.load_gather(table_v, [rows[p][sl]])
                    ews[p][sl] = ews[p][sl] * vals
                pltpu.async_copy(ews[p], acc_sh.at[cols[p]], sem_sc[p],
                                 add=True)
            scs[w] = pltpu.make_async_copy(ews[p], acc_sh.at[cols[p]],
                                           sem_sc[p])
        for w in list(scs):
            sc_desc = scs.pop(w)

            @pl.when(conds[w])
            def _(sc_desc=sc_desc):
                sc_desc.wait()
        plsc.subcore_barrier()
        pltpu.async_copy(tab4_hbm.at[jnp.minimum(ch + 1, 3)], table_v, sem_t)
        zb = ews[2]
        obase = ch * (2 * _NP) + cid * _NP + sid * _NCHUNK
        for q in range(_NCHUNK // _W):
            pltpu.sync_copy(acc_sh.at[pl.ds(sid * _NCHUNK + q * _W, _W)], zb)
            pltpu.sync_copy(zb, out_hbm.at[pl.ds(obase + q * _W, _W)])
        if _NCHUNK % _W:
            _r = _NCHUNK % _W
            _o = (_NCHUNK // _W) * _W
            pltpu.sync_copy(acc_sh.at[pl.ds(sid * _NCHUNK + _o, _r)],
                            zb.at[pl.ds(0, _r)])
            pltpu.sync_copy(zb.at[pl.ds(0, _r)],
                            out_hbm.at[pl.ds(obase + _o, _r)])
        zero_acc(zb)
        plsc.subcore_barrier()
        return 0

    lax.fori_loop(0, 4, chan_body, 0)
    wait_table()


def _d1_body(degp_ref, x_ref, dinv_ref, a1_ref):
    deg = degp_ref[0] + degp_ref[1] + 1.0
    dinv = lax.rsqrt(deg)
    dinv_ref[...] = dinv
    a1_ref[...] = dinv * x_ref[...]


_d1 = pl.pallas_call(
    _d1_body,
    out_shape=[jax.ShapeDtypeStruct(_NP2, jnp.float32),
               jax.ShapeDtypeStruct(_NP2, jnp.float32)],
)


def _d2_body(gp_ref, a1_ref, dinv_ref, c1_ref, d1_ref, w2_ref, a2t_ref):
    dinv = dinv_ref[...]
    s1 = dinv * (gp_ref[0] + gp_ref[1] + a1_ref[...])
    acc = [jnp.zeros(_NP2, jnp.float32) for _ in range(4)]
    for k in range(16):
        h = jnp.maximum(s1 * c1_ref[k] + d1_ref[k], 0.0)
        for j in range(4):
            acc[j] = acc[j] + h * w2_ref[k, j]
    for j in range(4):
        a2t_ref[j] = dinv * acc[j]


_d2 = pl.pallas_call(
    _d2_body,
    in_specs=[
        pl.BlockSpec(memory_space=pltpu.VMEM),
        pl.BlockSpec(memory_space=pltpu.VMEM),
        pl.BlockSpec(memory_space=pltpu.VMEM),
        pl.BlockSpec(memory_space=pltpu.SMEM),
        pl.BlockSpec(memory_space=pltpu.SMEM),
        pl.BlockSpec(memory_space=pltpu.SMEM),
    ],
    out_shape=jax.ShapeDtypeStruct((4,) + _NP2, jnp.float32),
)


def _d3_body(gp_ref, a2t_ref, dinv_ref, c2_ref, d2_ref, w3_ref, a3_ref):
    dinv = dinv_ref[...]
    u = jnp.zeros(_NP2, jnp.float32)
    for j in range(4):
        s2 = dinv * (gp_ref[j, 0] + gp_ref[j, 1] + a2t_ref[j])
        h2 = jnp.maximum(s2 * c2_ref[j] + d2_ref[j], 0.0)
        u = u + h2 * w3_ref[j]
    a3_ref[...] = dinv * u


_d3 = pl.pallas_call(
    _d3_body,
    in_specs=[
        pl.BlockSpec(memory_space=pltpu.VMEM),
        pl.BlockSpec(memory_space=pltpu.VMEM),
        pl.BlockSpec(memory_space=pltpu.VMEM),
        pl.BlockSpec(memory_space=pltpu.SMEM),
        pl.BlockSpec(memory_space=pltpu.SMEM),
        pl.BlockSpec(memory_space=pltpu.SMEM),
    ],
    out_shape=jax.ShapeDtypeStruct(_NP2, jnp.float32),
)


def _d4_body(gp_ref, a3_ref, dinv_ref, ab_ref, z_ref):
    s3 = dinv_ref[...] * (gp_ref[0] + gp_ref[1] + a3_ref[...])
    z_ref[...] = jax.nn.sigmoid(s3 * ab_ref[0] + ab_ref[1])


_d4 = pl.pallas_call(
    _d4_body,
    in_specs=[
        pl.BlockSpec(memory_space=pltpu.VMEM),
        pl.BlockSpec(memory_space=pltpu.VMEM),
        pl.BlockSpec(memory_space=pltpu.VMEM),
        pl.BlockSpec(memory_space=pltpu.SMEM),
    ],
    out_shape=jax.ShapeDtypeStruct(_NP2, jnp.float32),
)


def kernel(x, edge_attr, W1, b1, g1, be1, W2, b2, g2, be2, W3, b3, g3, be3,
           lw, lb, edge_index):
    f32 = jnp.float32
    ewp = edge_attr
    xp = jnp.concatenate([x[:, 0], jnp.zeros((_NP - _N,), f32)]).reshape(_NP2)

    k1 = f32(1.0) / jnp.sqrt(f32(1.0 + _EPS))
    c1 = W1[0] * k1 * g1
    d1 = b1 * k1 * g1 + be1
    c2 = k1 * g2
    d2 = b2 * k1 * g2 + be2
    w3 = W3[:, 0]
    A = k1 * g3[0] * lw[0, 0]
    B = (b3[0] * k1 * g3[0] + be3[0]) * lw[0, 0] + lb[0]
    ab = jnp.stack([A, B])

    degp = _deg_pass(edge_index, ewp).reshape((2,) + _NP2)
    dinv2, a1_2 = _d1(degp, xp)
    g1p = _agg_pass(a1_2.reshape(_NP), edge_index, ewp).reshape((2,) + _NP2)
    a2t = _d2(g1p, a1_2, dinv2, c1, d1, W2)
    g2p = _l2_pass(a2t.reshape(4, _NP), edge_index, ewp).reshape(
        (4, 2) + _NP2)
    a3_2 = _d3(g2p, a2t, dinv2, c2, d2, w3)
    g3p = _agg_pass(a3_2.reshape(_NP), edge_index, ewp).reshape((2,) + _NP2)
    z = _d4(g3p, a3_2, dinv2, ab)
    return z.reshape(_NP)[:_N].reshape(_N, 1)
